# baseline (speedup 1.0000x reference)
import functools

import jax
import jax.numpy as jnp
from jax import lax
from jax.experimental import pallas as pl
from jax.experimental.pallas import tpu as pltpu
from jax.experimental.pallas import tpu_sc as plsc

N = 10000
E = 320000
D_IN = 128
D_HID = 256
D_OUT = 128
N_EXP = 4

NC = 2
NS = 16
NW = NC * NS
CH = 80
NCH = 125
EPW = NCH * CH
EP = NW * EPW
NP = 10240
SLICE = NP // NS
ZR = 64
FH = 64

@functools.lru_cache(maxsize=None)
def _mesh():
    return plsc.VectorSubcoreMesh(
        core_axis_name="c", subcore_axis_name="s",
        num_cores=NC, num_subcores=NS,
    )


def _zero_fill_1d(buf, n):
    def body(i, _):
        buf[pl.ds(i * 16, 16)] = jnp.zeros((16,), jnp.float32)
        return 0
    lax.fori_loop(0, n // 16, body, 0)


@functools.lru_cache(maxsize=None)
def _make_deg_kernel():
    return functools.partial(
        pl.kernel,
        out_type=jax.ShapeDtypeStruct((NC, NP), jnp.float32),
        mesh=_mesh(),
        compiler_params=pltpu.CompilerParams(use_tc_tiling_on_sc=False),
        scratch_types=[
            pltpu.VMEM((NCH, CH), jnp.int32),
            pltpu.VMEM((CH,), jnp.float32),
            pltpu.VMEM((SLICE,), jnp.float32),
            pltpu.VMEM_SHARED((NP,), jnp.float32),
        ],
    )(_deg_body)


def _deg_body(dst_hbm, out_hbm, idx_v, ones_v, zero_v, acc_sh):
    cid = lax.axis_index("c")
    sid = lax.axis_index("s")
    wid = cid * NS + sid

    _zero_fill_1d(zero_v, SLICE)
    for i in range(CH // 16):
        ones_v[pl.ds(i * 16, 16)] = jnp.ones((16,), jnp.float32)
    pltpu.sync_copy(zero_v, acc_sh.at[pl.ds(sid * SLICE, SLICE)])
    pltpu.sync_copy(dst_hbm.at[wid], idx_v)
    plsc.subcore_barrier()

    def body(c, _):
        pltpu.sync_copy(ones_v, acc_sh.at[idx_v.at[c]], add=True)
        return 0
    lax.fori_loop(0, NCH, body, 0)

    plsc.subcore_barrier()
    pltpu.sync_copy(
        acc_sh.at[pl.ds(sid * SLICE, SLICE)],
        out_hbm.at[cid, pl.ds(sid * SLICE, SLICE)],
    )


@functools.lru_cache(maxsize=None)
def _make_prop_kernel(T, W):
    @functools.partial(
        pl.kernel,
        out_type=jax.ShapeDtypeStruct((NC, T * NP, W), jnp.float32),
        mesh=_mesh(),
        compiler_params=pltpu.CompilerParams(use_tc_tiling_on_sc=False),
        scratch_types=[
            pltpu.VMEM((NCH, CH), jnp.int32),
            pltpu.VMEM((NCH, CH), jnp.int32),
            pltpu.VMEM((CH, W), jnp.float32),
            pltpu.VMEM((CH, W), jnp.float32),
            pltpu.VMEM((ZR, W), jnp.float32),
            pltpu.VMEM_SHARED((NP, W), jnp.float32),
            pltpu.SemaphoreType.DMA,
            pltpu.SemaphoreType.DMA,
            pltpu.SemaphoreType.DMA,
            pltpu.SemaphoreType.DMA,
        ],
    )
    def _prop(src_hbm, dst_hbm, *rest):
        tables = rest[:T]
        out_hbm = rest[T]
        (src_v, dst_v, rows0_v, rows1_v, zero_v, acc_sh,
         sem0, sem1, ssem0, ssem1) = rest[T + 1:]
        cid = lax.axis_index("c")
        sid = lax.axis_index("s")
        wid = cid * NS + sid

        def zbody(r, _):
            for c in range(W // 16):
                zero_v[r, pl.ds(c * 16, 16)] = jnp.zeros((16,), jnp.float32)
            return 0
        lax.fori_loop(0, ZR, zbody, 0)

        pltpu.sync_copy(src_hbm.at[wid], src_v)
        pltpu.sync_copy(dst_hbm.at[wid], dst_v)

        for t in range(T):
            tbl = tables[t]
            for bz in range(SLICE // ZR):
                pltpu.sync_copy(
                    zero_v, acc_sh.at[pl.ds(sid * SLICE + bz * ZR, ZR)]
                )
            plsc.subcore_barrier()

            def startg(buf, sem, c):
                pltpu.async_copy(tbl.at[src_v.at[c]], buf, sem)

            def waitg(buf, sem, c):
                pltpu.make_async_copy(tbl.at[src_v.at[c]], buf, sem).wait()

            def starts(buf, sem, c):
                pltpu.async_copy(buf, acc_sh.at[dst_v.at[c]], sem, add=True)

            def waits(buf, sem, c):
                pltpu.make_async_copy(buf, acc_sh.at[dst_v.at[c]], sem).wait()

            startg(rows0_v, sem0, 0)
            waitg(rows0_v, sem0, 0)
            starts(rows0_v, ssem0, 0)
            startg(rows1_v, sem1, 1)

            def body(i, _):
                c = 2 * i + 1
                waitg(rows1_v, sem1, c)
                starts(rows1_v, ssem1, c)
                waits(rows0_v, ssem0, c - 1)
                startg(rows0_v, sem0, c + 1)
                waitg(rows0_v, sem0, c + 1)
                starts(rows0_v, ssem0, c + 1)
                waits(rows1_v, ssem1, c)
                startg(rows1_v, sem1, c + 2)
                return 0
            lax.fori_loop(0, (NCH - 3) // 2, body, 0)
            waitg(rows1_v, sem1, NCH - 2)
            starts(rows1_v, ssem1, NCH - 2)
            waits(rows0_v, ssem0, NCH - 3)
            startg(rows0_v, sem0, NCH - 1)
            waitg(rows0_v, sem0, NCH - 1)
            starts(rows0_v, ssem0, NCH - 1)
            waits(rows1_v, ssem1, NCH - 2)
            waits(rows0_v, ssem0, NCH - 1)

            plsc.subcore_barrier()
            pltpu.sync_copy(
                acc_sh.at[pl.ds(sid * SLICE, SLICE)],
                out_hbm.at[cid, pl.ds(t * NP + sid * SLICE, SLICE)],
            )
    return _prop


BLK = 512
NB = NP // BLK


def _isq_of(deg2_blk):
    deg = deg2_blk[0] + deg2_blk[1]
    return lax.rsqrt(jnp.maximum(deg, 1.0))


def _prep_body(deg2_ref, x_ref, xsa_ref, xsb_ref):
    isq = _isq_of(deg2_ref[...])
    xs = x_ref[...] * isq[:, None]
    xsa_ref[...] = xs[:, :FH]
    xsb_ref[...] = xs[:, FH:]


def _prep_call(deg2, x):
    return pl.pallas_call(
        _prep_body,
        grid=(NB,),
        in_specs=[
            pl.BlockSpec((NC, BLK), lambda i: (0, i)),
            pl.BlockSpec((BLK, D_IN), lambda i: (i, 0)),
        ],
        out_specs=[pl.BlockSpec((BLK, FH), lambda i: (i, 0))] * 2,
        out_shape=[jax.ShapeDtypeStruct((N, FH), jnp.float32)] * 2,
    )(deg2, x)


def _experts_body(deg2_ref, s1_ref, W1_ref, b1_ref, W2_ref, *out_refs):
    isq = _isq_of(deg2_ref[...])
    s1 = s1_ref[0] + s1_ref[1]
    p1 = jnp.concatenate([s1[0], s1[1]], axis=1) * isq[:, None]
    for e in range(N_EXP):
        h = jnp.dot(p1, W1_ref[e], preferred_element_type=jnp.float32)
        h = jnp.maximum(h + b1_ref[e][None, :], 0.0)
        g = jnp.dot(h, W2_ref[e], preferred_element_type=jnp.float32)
        out_refs[e][...] = g * isq[:, None]


def _experts_call(deg2, s1, W1, b1, W2):
    return pl.pallas_call(
        _experts_body,
        grid=(NB,),
        in_specs=[
            pl.BlockSpec((NC, BLK), lambda i: (0, i)),
            pl.BlockSpec((NC, 2, BLK, FH), lambda i: (0, 0, i, 0)),
            pl.BlockSpec((N_EXP, D_IN, D_HID), lambda i: (0, 0, 0)),
            pl.BlockSpec((N_EXP, D_HID), lambda i: (0, 0)),
            pl.BlockSpec((N_EXP, D_HID, D_OUT), lambda i: (0, 0, 0)),
        ],
        out_specs=[pl.BlockSpec((BLK, D_OUT), lambda i: (i, 0))] * N_EXP,
        out_shape=[jax.ShapeDtypeStruct((N, D_OUT), jnp.float32)] * N_EXP,
    )(deg2, s1, W1, b1, W2)


def _final_body(deg2_ref, x_ref, wg_ref, thr_ref, mask_ref, b2_ref, s2_ref,
                out_ref):
    isq = _isq_of(deg2_ref[...])
    x = x_ref[...]
    logits = jnp.dot(x, wg_ref[...], preferred_element_type=jnp.float32)
    m = jnp.max(logits, axis=1, keepdims=True)
    ex = jnp.exp(logits - m)
    soft = ex / jnp.sum(ex, axis=1, keepdims=True)
    hard = 0.5 * (jnp.sign(logits - thr_ref[0][None, :]) + 1.0)
    gates = soft * hard * mask_ref[0][None, :]
    gates = gates / (jnp.sum(gates, axis=1, keepdims=True) + 1e-10)
    acc = jnp.dot(gates, b2_ref[...], preferred_element_type=jnp.float32)
    s2 = s2_ref[0] + s2_ref[1]
    total = None
    for e in range(N_EXP):
        term = gates[:, e:e + 1] * s2[e]
        total = term if total is None else total + term
    out_ref[...] = total * isq[:, None] + acc


def _final_call(deg2, x, w_gate, thr, mask, b2, s2):
    return pl.pallas_call(
        _final_body,
        grid=(NB,),
        in_specs=[
            pl.BlockSpec((NC, BLK), lambda i: (0, i)),
            pl.BlockSpec((BLK, D_IN), lambda i: (i, 0)),
            pl.BlockSpec((D_IN, N_EXP), lambda i: (0, 0)),
            pl.BlockSpec((1, N_EXP), lambda i: (0, 0)),
            pl.BlockSpec((1, N_EXP), lambda i: (0, 0)),
            pl.BlockSpec((N_EXP, D_OUT), lambda i: (0, 0)),
            pl.BlockSpec((NC, N_EXP, BLK, D_OUT), lambda i: (0, 0, i, 0)),
        ],
        out_specs=pl.BlockSpec((BLK, D_OUT), lambda i: (i, 0)),
        out_shape=jax.ShapeDtypeStruct((N, D_OUT), jnp.float32),
    )(deg2, x, w_gate, thr, mask, b2, s2)


def kernel(x, edge_index, w_gate, gate_threshold, W1, b1, W2, b2,
           experts_mask):
    src2 = edge_index[0].astype(jnp.int32).reshape(NW, NCH, CH)
    dst2 = edge_index[1].astype(jnp.int32).reshape(NW, NCH, CH)

    deg2 = _make_deg_kernel()(dst2)
    xsa, xsb = _prep_call(deg2, x)
    s1 = _make_prop_kernel(2, FH)(src2, dst2, xsa, xsb)
    s1 = s1.reshape(NC, 2, NP, FH)
    g = _experts_call(deg2, s1, W1, b1, W2)
    s2 = _make_prop_kernel(N_EXP, D_OUT)(src2, dst2, *g)
    s2 = s2.reshape(NC, N_EXP, NP, D_OUT)
    return _final_call(
        deg2, x, w_gate,
        gate_threshold.reshape(1, N_EXP), experts_mask.reshape(1, N_EXP),
        b2, s2,
    )


# speedup vs baseline: 1.0031x; 1.0031x over previous
import functools

import jax
import jax.numpy as jnp
from jax import lax
from jax.experimental import pallas as pl
from jax.experimental.pallas import tpu as pltpu
from jax.experimental.pallas import tpu_sc as plsc

N = 10000
E = 320000
D_IN = 128
D_HID = 256
D_OUT = 128
N_EXP = 4

NC = 2
NS = 16
NW = NC * NS
CH = 80
NCH = 125
EPW = NCH * CH
EP = NW * EPW
NP = 10240
SLICE = NP // NS
ZR = 64
FH = 64

@functools.lru_cache(maxsize=None)
def _mesh():
    return plsc.VectorSubcoreMesh(
        core_axis_name="c", subcore_axis_name="s",
        num_cores=NC, num_subcores=NS,
    )


def _zero_fill_1d(buf, n):
    def body(i, _):
        buf[pl.ds(i * 16, 16)] = jnp.zeros((16,), jnp.float32)
        return 0
    lax.fori_loop(0, n // 16, body, 0)


@functools.lru_cache(maxsize=None)
def _make_deg_kernel():
    return functools.partial(
        pl.kernel,
        out_type=jax.ShapeDtypeStruct((NC, NP), jnp.float32),
        mesh=_mesh(),
        compiler_params=pltpu.CompilerParams(use_tc_tiling_on_sc=False),
        scratch_types=[
            pltpu.VMEM((NCH, CH), jnp.int32),
            pltpu.VMEM((CH,), jnp.float32),
            pltpu.VMEM((SLICE,), jnp.float32),
            pltpu.VMEM_SHARED((NP,), jnp.float32),
        ],
    )(_deg_body)


def _deg_body(dst_hbm, out_hbm, idx_v, ones_v, zero_v, acc_sh):
    cid = lax.axis_index("c")
    sid = lax.axis_index("s")
    wid = cid * NS + sid

    _zero_fill_1d(zero_v, SLICE)
    for i in range(CH // 16):
        ones_v[pl.ds(i * 16, 16)] = jnp.ones((16,), jnp.float32)
    pltpu.sync_copy(zero_v, acc_sh.at[pl.ds(sid * SLICE, SLICE)])
    pltpu.sync_copy(dst_hbm.at[wid], idx_v)
    plsc.subcore_barrier()

    def body(c, _):
        pltpu.sync_copy(ones_v, acc_sh.at[idx_v.at[c]], add=True)
        return 0
    lax.fori_loop(0, NCH, body, 0)

    plsc.subcore_barrier()
    pltpu.sync_copy(
        acc_sh.at[pl.ds(sid * SLICE, SLICE)],
        out_hbm.at[cid, pl.ds(sid * SLICE, SLICE)],
    )


@functools.lru_cache(maxsize=None)
def _make_prop_kernel(T, W):
    @functools.partial(
        pl.kernel,
        out_type=jax.ShapeDtypeStruct((NC, T * NP, W), jnp.float32),
        mesh=_mesh(),
        compiler_params=pltpu.CompilerParams(use_tc_tiling_on_sc=False),
        scratch_types=[
            pltpu.VMEM((NCH, CH), jnp.int32),
            pltpu.VMEM((NCH, CH), jnp.int32),
            pltpu.VMEM((CH, W), jnp.float32),
            pltpu.VMEM((CH, W), jnp.float32),
            pltpu.VMEM((ZR, W), jnp.float32),
            pltpu.VMEM_SHARED((NP, W), jnp.float32),
            pltpu.SemaphoreType.DMA,
            pltpu.SemaphoreType.DMA,
        ],
    )
    def _prop(src_hbm, dst_hbm, *rest):
        tables = rest[:T]
        out_hbm = rest[T]
        (src_v, dst_v, rows0_v, rows1_v, zero_v, acc_sh,
         sem0, sem1) = rest[T + 1:]
        cid = lax.axis_index("c")
        sid = lax.axis_index("s")
        wid = cid * NS + sid

        def zbody(r, _):
            for c in range(W // 16):
                zero_v[r, pl.ds(c * 16, 16)] = jnp.zeros((16,), jnp.float32)
            return 0
        lax.fori_loop(0, ZR, zbody, 0)

        pltpu.sync_copy(src_hbm.at[wid], src_v)
        pltpu.sync_copy(dst_hbm.at[wid], dst_v)

        for t in range(T):
            tbl = tables[t]
            for bz in range(SLICE // ZR):
                pltpu.sync_copy(
                    zero_v, acc_sh.at[pl.ds(sid * SLICE + bz * ZR, ZR)]
                )
            plsc.subcore_barrier()

            pltpu.async_copy(tbl.at[src_v.at[0]], rows0_v, sem0)

            def body(i, _):
                c0 = 2 * i
                c1 = 2 * i + 1
                pltpu.make_async_copy(
                    tbl.at[src_v.at[c0]], rows0_v, sem0).wait()
                pltpu.async_copy(tbl.at[src_v.at[c1]], rows1_v, sem1)
                pltpu.sync_copy(rows0_v, acc_sh.at[dst_v.at[c0]], add=True)
                pltpu.make_async_copy(
                    tbl.at[src_v.at[c1]], rows1_v, sem1).wait()
                pltpu.async_copy(tbl.at[src_v.at[c1 + 1]], rows0_v, sem0)
                pltpu.sync_copy(rows1_v, acc_sh.at[dst_v.at[c1]], add=True)
                return 0
            lax.fori_loop(0, NCH // 2, body, 0)
            pltpu.make_async_copy(
                tbl.at[src_v.at[NCH - 1]], rows0_v, sem0).wait()
            pltpu.sync_copy(
                rows0_v, acc_sh.at[dst_v.at[NCH - 1]], add=True)

            plsc.subcore_barrier()
            pltpu.sync_copy(
                acc_sh.at[pl.ds(sid * SLICE, SLICE)],
                out_hbm.at[cid, pl.ds(t * NP + sid * SLICE, SLICE)],
            )
    return _prop


BLK = 512
NB = NP // BLK


def _isq_of(deg2_blk):
    deg = deg2_blk[0] + deg2_blk[1]
    return lax.rsqrt(jnp.maximum(deg, 1.0))


def _prep_body(deg2_ref, x_ref, xsa_ref, xsb_ref):
    isq = _isq_of(deg2_ref[...])
    xs = x_ref[...] * isq[:, None]
    xsa_ref[...] = xs[:, :FH]
    xsb_ref[...] = xs[:, FH:]


def _prep_call(deg2, x):
    return pl.pallas_call(
        _prep_body,
        grid=(NB,),
        in_specs=[
            pl.BlockSpec((NC, BLK), lambda i: (0, i)),
            pl.BlockSpec((BLK, D_IN), lambda i: (i, 0)),
        ],
        out_specs=[pl.BlockSpec((BLK, FH), lambda i: (i, 0))] * 2,
        out_shape=[jax.ShapeDtypeStruct((N, FH), jnp.float32)] * 2,
    )(deg2, x)


def _experts_body(deg2_ref, s1_ref, W1_ref, b1_ref, W2_ref, *out_refs):
    isq = _isq_of(deg2_ref[...])
    s1 = s1_ref[0] + s1_ref[1]
    p1 = jnp.concatenate([s1[0], s1[1]], axis=1) * isq[:, None]
    for e in range(N_EXP):
        h = jnp.dot(p1, W1_ref[e], preferred_element_type=jnp.float32)
        h = jnp.maximum(h + b1_ref[e][None, :], 0.0)
        g = jnp.dot(h, W2_ref[e], preferred_element_type=jnp.float32)
        out_refs[e][...] = g * isq[:, None]


def _experts_call(deg2, s1, W1, b1, W2):
    return pl.pallas_call(
        _experts_body,
        grid=(NB,),
        in_specs=[
            pl.BlockSpec((NC, BLK), lambda i: (0, i)),
            pl.BlockSpec((NC, 2, BLK, FH), lambda i: (0, 0, i, 0)),
            pl.BlockSpec((N_EXP, D_IN, D_HID), lambda i: (0, 0, 0)),
            pl.BlockSpec((N_EXP, D_HID), lambda i: (0, 0)),
            pl.BlockSpec((N_EXP, D_HID, D_OUT), lambda i: (0, 0, 0)),
        ],
        out_specs=[pl.BlockSpec((BLK, D_OUT), lambda i: (i, 0))] * N_EXP,
        out_shape=[jax.ShapeDtypeStruct((N, D_OUT), jnp.float32)] * N_EXP,
    )(deg2, s1, W1, b1, W2)


def _final_body(deg2_ref, x_ref, wg_ref, thr_ref, mask_ref, b2_ref, s2_ref,
                out_ref):
    isq = _isq_of(deg2_ref[...])
    x = x_ref[...]
    logits = jnp.dot(x, wg_ref[...], preferred_element_type=jnp.float32)
    m = jnp.max(logits, axis=1, keepdims=True)
    ex = jnp.exp(logits - m)
    soft = ex / jnp.sum(ex, axis=1, keepdims=True)
    hard = 0.5 * (jnp.sign(logits - thr_ref[0][None, :]) + 1.0)
    gates = soft * hard * mask_ref[0][None, :]
    gates = gates / (jnp.sum(gates, axis=1, keepdims=True) + 1e-10)
    acc = jnp.dot(gates, b2_ref[...], preferred_element_type=jnp.float32)
    s2 = s2_ref[0] + s2_ref[1]
    total = None
    for e in range(N_EXP):
        term = gates[:, e:e + 1] * s2[e]
        total = term if total is None else total + term
    out_ref[...] = total * isq[:, None] + acc


def _final_call(deg2, x, w_gate, thr, mask, b2, s2):
    return pl.pallas_call(
        _final_body,
        grid=(NB,),
        in_specs=[
            pl.BlockSpec((NC, BLK), lambda i: (0, i)),
            pl.BlockSpec((BLK, D_IN), lambda i: (i, 0)),
            pl.BlockSpec((D_IN, N_EXP), lambda i: (0, 0)),
            pl.BlockSpec((1, N_EXP), lambda i: (0, 0)),
            pl.BlockSpec((1, N_EXP), lambda i: (0, 0)),
            pl.BlockSpec((N_EXP, D_OUT), lambda i: (0, 0)),
            pl.BlockSpec((NC, N_EXP, BLK, D_OUT), lambda i: (0, 0, i, 0)),
        ],
        out_specs=pl.BlockSpec((BLK, D_OUT), lambda i: (i, 0)),
        out_shape=jax.ShapeDtypeStruct((N, D_OUT), jnp.float32),
    )(deg2, x, w_gate, thr, mask, b2, s2)


def kernel(x, edge_index, w_gate, gate_threshold, W1, b1, W2, b2,
           experts_mask):
    src2 = edge_index[0].astype(jnp.int32).reshape(NW, NCH, CH)
    dst2 = edge_index[1].astype(jnp.int32).reshape(NW, NCH, CH)

    deg2 = _make_deg_kernel()(dst2)
    xsa, xsb = _prep_call(deg2, x)
    s1 = _make_prop_kernel(2, FH)(src2, dst2, xsa, xsb)
    s1 = s1.reshape(NC, 2, NP, FH)
    g = _experts_call(deg2, s1, W1, b1, W2)
    s2 = _make_prop_kernel(N_EXP, D_OUT)(src2, dst2, *g)
    s2 = s2.reshape(NC, N_EXP, NP, D_OUT)
    return _final_call(
        deg2, x, w_gate,
        gate_threshold.reshape(1, N_EXP), experts_mask.reshape(1, N_EXP),
        b2, s2,
    )


# speedup vs baseline: 1.1146x; 1.1111x over previous
import functools

import jax
import jax.numpy as jnp
from jax import lax
from jax.experimental import pallas as pl
from jax.experimental.pallas import tpu as pltpu
from jax.experimental.pallas import tpu_sc as plsc

N = 10000
E = 320000
D_IN = 128
D_HID = 256
D_OUT = 128
N_EXP = 4

NC = 2
NS = 16
NW = NC * NS
CH = 80
NCH = 125
EPW = NCH * CH
EP = NW * EPW
NP = 10240
SLICE = NP // NS
ZR = 64
FH = 64

@functools.lru_cache(maxsize=None)
def _mesh():
    return plsc.VectorSubcoreMesh(
        core_axis_name="c", subcore_axis_name="s",
        num_cores=NC, num_subcores=NS,
    )


def _zero_fill_1d(buf, n):
    def body(i, _):
        buf[pl.ds(i * 16, 16)] = jnp.zeros((16,), jnp.float32)
        return 0
    lax.fori_loop(0, n // 16, body, 0)


@functools.lru_cache(maxsize=None)
def _make_deg_kernel():
    return functools.partial(
        pl.kernel,
        out_type=jax.ShapeDtypeStruct((NC, NP), jnp.float32),
        mesh=_mesh(),
        compiler_params=pltpu.CompilerParams(use_tc_tiling_on_sc=False),
        scratch_types=[
            pltpu.VMEM((NCH, CH), jnp.int32),
            pltpu.VMEM((CH,), jnp.float32),
            pltpu.VMEM((SLICE,), jnp.float32),
            pltpu.VMEM_SHARED((NP,), jnp.float32),
        ],
    )(_deg_body)


def _deg_body(dst_hbm, out_hbm, idx_v, ones_v, zero_v, acc_sh):
    cid = lax.axis_index("c")
    sid = lax.axis_index("s")
    wid = cid * NS + sid

    _zero_fill_1d(zero_v, SLICE)
    for i in range(CH // 16):
        ones_v[pl.ds(i * 16, 16)] = jnp.ones((16,), jnp.float32)
    pltpu.sync_copy(zero_v, acc_sh.at[pl.ds(sid * SLICE, SLICE)])
    pltpu.sync_copy(dst_hbm.at[wid], idx_v)
    plsc.subcore_barrier()

    def body(c, _):
        pltpu.sync_copy(ones_v, acc_sh.at[idx_v.at[c]], add=True)
        return 0
    lax.fori_loop(0, NCH, body, 0)

    plsc.subcore_barrier()
    pltpu.sync_copy(
        acc_sh.at[pl.ds(sid * SLICE, SLICE)],
        out_hbm.at[cid, pl.ds(sid * SLICE, SLICE)],
    )


@functools.lru_cache(maxsize=None)
def _make_prop_kernel(T, W):
    @functools.partial(
        pl.kernel,
        out_type=jax.ShapeDtypeStruct((NC, T * NP, W), jnp.float32),
        mesh=_mesh(),
        compiler_params=pltpu.CompilerParams(use_tc_tiling_on_sc=False),
        scratch_types=[
            pltpu.VMEM((NCH, CH), jnp.int32),
            pltpu.VMEM((NCH, CH), jnp.int32),
            pltpu.VMEM((CH, W), jnp.float32),
            pltpu.VMEM((CH, W), jnp.float32),
            pltpu.VMEM((ZR, W), jnp.float32),
            pltpu.VMEM_SHARED((NP, W), jnp.float32),
            pltpu.SemaphoreType.DMA,
            pltpu.SemaphoreType.DMA,
        ],
    )
    def _prop(src_hbm, dst_hbm, *rest):
        tables = rest[:T]
        out_hbm = rest[T]
        (src_v, dst_v, rows0_v, rows1_v, zero_v, acc_sh,
         sem0, sem1) = rest[T + 1:]
        cid = lax.axis_index("c")
        sid = lax.axis_index("s")
        wid = cid * NS + sid

        def zbody(r, _):
            for c in range(W // 16):
                zero_v[r, pl.ds(c * 16, 16)] = jnp.zeros((16,), jnp.float32)
            return 0
        lax.fori_loop(0, ZR, zbody, 0)

        pltpu.sync_copy(src_hbm.at[wid], src_v)
        pltpu.sync_copy(dst_hbm.at[wid], dst_v)

        for t in range(T):
            tbl = tables[t]
            for bz in range(SLICE // ZR):
                pltpu.sync_copy(
                    zero_v, acc_sh.at[pl.ds(sid * SLICE + bz * ZR, ZR)]
                )
            plsc.subcore_barrier()

            pltpu.async_copy(tbl.at[src_v.at[0]], rows0_v, sem0)

            def body(i, _):
                c0 = 2 * i
                c1 = 2 * i + 1
                pltpu.make_async_copy(
                    tbl.at[src_v.at[c0]], rows0_v, sem0).wait()
                pltpu.async_copy(tbl.at[src_v.at[c1]], rows1_v, sem1)
                pltpu.sync_copy(rows0_v, acc_sh.at[dst_v.at[c0]], add=True)
                pltpu.make_async_copy(
                    tbl.at[src_v.at[c1]], rows1_v, sem1).wait()
                pltpu.async_copy(tbl.at[src_v.at[c1 + 1]], rows0_v, sem0)
                pltpu.sync_copy(rows1_v, acc_sh.at[dst_v.at[c1]], add=True)
                return 0
            lax.fori_loop(0, NCH // 2, body, 0)
            pltpu.make_async_copy(
                tbl.at[src_v.at[NCH - 1]], rows0_v, sem0).wait()
            pltpu.sync_copy(
                rows0_v, acc_sh.at[dst_v.at[NCH - 1]], add=True)

            plsc.subcore_barrier()
            pltpu.sync_copy(
                acc_sh.at[pl.ds(sid * SLICE, SLICE)],
                out_hbm.at[cid, pl.ds(t * NP + sid * SLICE, SLICE)],
            )
    return _prop


BLK = 512
NB = NP // BLK


def _isq_of(deg2_blk):
    deg = deg2_blk[0] + deg2_blk[1]
    return lax.rsqrt(jnp.maximum(deg, 1.0))


def _prep_body(deg2_ref, x_ref, xs_ref):
    isq = _isq_of(deg2_ref[...])
    xs_ref[...] = x_ref[...] * isq[:, None]


def _prep_call(deg2, x):
    return pl.pallas_call(
        _prep_body,
        grid=(NB,),
        in_specs=[
            pl.BlockSpec((NC, BLK), lambda i: (0, i)),
            pl.BlockSpec((BLK, D_IN), lambda i: (i, 0)),
        ],
        out_specs=pl.BlockSpec((BLK, D_IN), lambda i: (i, 0)),
        out_shape=jax.ShapeDtypeStruct((N, D_IN), jnp.float32),
    )(deg2, x)


def _experts_body(deg2_ref, s1_ref, W1_ref, b1_ref, W2_ref, *out_refs):
    isq = _isq_of(deg2_ref[...])
    p1 = (s1_ref[0] + s1_ref[1]) * isq[:, None]
    for e in range(N_EXP):
        h = jnp.dot(p1, W1_ref[e], preferred_element_type=jnp.float32)
        h = jnp.maximum(h + b1_ref[e][None, :], 0.0)
        g = jnp.dot(h, W2_ref[e], preferred_element_type=jnp.float32)
        out_refs[e][...] = g * isq[:, None]


def _experts_call(deg2, s1, W1, b1, W2):
    return pl.pallas_call(
        _experts_body,
        grid=(NB,),
        in_specs=[
            pl.BlockSpec((NC, BLK), lambda i: (0, i)),
            pl.BlockSpec((NC, BLK, D_IN), lambda i: (0, i, 0)),
            pl.BlockSpec((N_EXP, D_IN, D_HID), lambda i: (0, 0, 0)),
            pl.BlockSpec((N_EXP, D_HID), lambda i: (0, 0)),
            pl.BlockSpec((N_EXP, D_HID, D_OUT), lambda i: (0, 0, 0)),
        ],
        out_specs=[pl.BlockSpec((BLK, D_OUT), lambda i: (i, 0))] * N_EXP,
        out_shape=[jax.ShapeDtypeStruct((N, D_OUT), jnp.float32)] * N_EXP,
    )(deg2, s1, W1, b1, W2)


def _final_body(deg2_ref, x_ref, wg_ref, thr_ref, mask_ref, b2_ref, s2_ref,
                out_ref):
    isq = _isq_of(deg2_ref[...])
    x = x_ref[...]
    logits = jnp.dot(x, wg_ref[...], preferred_element_type=jnp.float32)
    m = jnp.max(logits, axis=1, keepdims=True)
    ex = jnp.exp(logits - m)
    soft = ex / jnp.sum(ex, axis=1, keepdims=True)
    hard = 0.5 * (jnp.sign(logits - thr_ref[0][None, :]) + 1.0)
    gates = soft * hard * mask_ref[0][None, :]
    gates = gates / (jnp.sum(gates, axis=1, keepdims=True) + 1e-10)
    acc = jnp.dot(gates, b2_ref[...], preferred_element_type=jnp.float32)
    s2 = s2_ref[0] + s2_ref[1]
    total = None
    for e in range(N_EXP):
        term = gates[:, e:e + 1] * s2[e]
        total = term if total is None else total + term
    out_ref[...] = total * isq[:, None] + acc


def _final_call(deg2, x, w_gate, thr, mask, b2, s2):
    return pl.pallas_call(
        _final_body,
        grid=(NB,),
        in_specs=[
            pl.BlockSpec((NC, BLK), lambda i: (0, i)),
            pl.BlockSpec((BLK, D_IN), lambda i: (i, 0)),
            pl.BlockSpec((D_IN, N_EXP), lambda i: (0, 0)),
            pl.BlockSpec((1, N_EXP), lambda i: (0, 0)),
            pl.BlockSpec((1, N_EXP), lambda i: (0, 0)),
            pl.BlockSpec((N_EXP, D_OUT), lambda i: (0, 0)),
            pl.BlockSpec((NC, N_EXP, BLK, D_OUT), lambda i: (0, 0, i, 0)),
        ],
        out_specs=pl.BlockSpec((BLK, D_OUT), lambda i: (i, 0)),
        out_shape=jax.ShapeDtypeStruct((N, D_OUT), jnp.float32),
    )(deg2, x, w_gate, thr, mask, b2, s2)


def kernel(x, edge_index, w_gate, gate_threshold, W1, b1, W2, b2,
           experts_mask):
    src2 = edge_index[0].astype(jnp.int32).reshape(NW, NCH, CH)
    dst2 = edge_index[1].astype(jnp.int32).reshape(NW, NCH, CH)

    deg2 = _make_deg_kernel()(dst2)
    xs = _prep_call(deg2, x)
    s1 = _make_prop_kernel(1, D_IN)(src2, dst2, xs)
    s1 = s1.reshape(NC, NP, D_IN)
    g = _experts_call(deg2, s1, W1, b1, W2)
    s2 = _make_prop_kernel(N_EXP, D_OUT)(src2, dst2, *g)
    s2 = s2.reshape(NC, N_EXP, NP, D_OUT)
    return _final_call(
        deg2, x, w_gate,
        gate_threshold.reshape(1, N_EXP), experts_mask.reshape(1, N_EXP),
        b2, s2,
    )
